# Initial kernel scaffold; baseline (speedup 1.0000x reference)
import jax
import jax.numpy as jnp
from jax.experimental import pallas as pl


def kernel(input_ids_BL, gene_table, pos_table, ln_gamma, ln_beta):
    raise NotImplementedError("write your pallas kernel here")


# speedup vs baseline: 2.6714x; 2.6714x over previous
import functools

import jax
import jax.numpy as jnp
from jax import lax
from jax.experimental import pallas as pl
from jax.experimental.pallas import tpu as pltpu
from jax.experimental.pallas import tpu_sc as plsc

B = 4096
L = 200
D = 64
EPS = 1e-5

NC = 2
NS = 16
NW = NC * NS

BPW = B // NW
BB = 16
LL = 25
NBI = BPW // BB
NLI = L // LL
ROWS = BB * LL
GCH = 100
NGC = ROWS // GCH

MAGIC = 0x5F3759DF


def _rsqrt(v):
    i = plsc.bitcast(v, jnp.int32)
    i = MAGIC - lax.shift_right_logical(i, 1)
    y = plsc.bitcast(i, jnp.float32)
    hv = 0.5 * v
    y = y * (1.5 - hv * y * y)
    y = y * (1.5 - hv * y * y)
    y = y * (1.5 - hv * y * y)
    return y


def _sc_kernel(ids_hbm, table_hbm, pos_hbm, gam_hbm, bet_hbm, out_hbm,
               idx_v, inbuf, outbuf, pos_v, gam_v, bet_v, sem):
    cid = lax.axis_index("c")
    sid = lax.axis_index("s")
    wid = cid * NS + sid

    pltpu.sync_copy(pos_hbm.at[pl.ds(0, L * D)], pos_v)
    pltpu.sync_copy(gam_hbm, gam_v)
    pltpu.sync_copy(bet_hbm, bet_v)

    iota = lax.iota(jnp.int32, 16)
    cols = [iota + 16 * k for k in range(4)]
    g = [gam_v[pl.ds(16 * k, 16)] for k in range(4)]
    bta = [bet_v[pl.ds(16 * k, 16)] for k in range(4)]

    @pl.loop(0, NBI)
    def _bb_loop(bb):
        @pl.loop(0, NLI)
        def _lb_loop(lb):
            blk = (wid * NBI + bb) * NLI + lb
            pltpu.sync_copy(ids_hbm.at[pl.ds(blk * NGC, NGC), :], idx_v)
            copies = []
            for j in range(NGC):
                copies.append(pltpu.async_copy(
                    table_hbm.at[idx_v.at[j]],
                    inbuf.at[pl.ds(j * GCH, GCH), :], sem))
            for c in copies:
                c.wait()

            @pl.loop(0, LL)
            def _li_loop(li):
                l = lb * LL + li
                p = [pos_v[pl.ds(l * D + 16 * k, 16)] for k in range(4)]
                for bi in range(BB):
                    row = bi * LL + li
                    x = [inbuf[row, pl.ds(16 * k, 16)] + p[k]
                         for k in range(4)]
                    tot = (x[0] + x[1]) + (x[2] + x[3])
                    q = ((x[0] * x[0] + x[1] * x[1])
                         + (x[2] * x[2] + x[3] * x[3]))
                    sv = jnp.full((16,), jnp.sum(tot))
                    qv = jnp.full((16,), jnp.sum(q))
                    mean = sv * (1.0 / D)
                    var = qv * (1.0 / D) - mean * mean
                    rstd = _rsqrt(var + EPS)
                    base = row * D
                    for k in range(4):
                        y = (x[k] - mean) * rstd * g[k] + bta[k]
                        outbuf[pl.ds(base + 16 * k, 16)] = y

            for bi in range(BB):
                dst = (wid * BPW + bb * BB + bi) * (L * D) + lb * (LL * D)
                pltpu.sync_copy(outbuf.at[pl.ds(bi * LL * D, LL * D)],
                                out_hbm.at[pl.ds(dst, LL * D)])


@jax.jit
def kernel(input_ids_BL, gene_table, pos_table, ln_gamma, ln_beta):
    ids = input_ids_BL.astype(jnp.int32)
    ids_prep = (ids.reshape(NW, NBI, BB, NLI, LL)
                .transpose(0, 1, 3, 2, 4)
                .reshape(-1, GCH))
    pos_flat = pos_table.reshape(-1)

    mesh = plsc.VectorSubcoreMesh(core_axis_name="c", subcore_axis_name="s",
                                  num_cores=NC, num_subcores=NS)
    out_flat = pl.kernel(
        _sc_kernel,
        out_type=jax.ShapeDtypeStruct((B * L * D,), jnp.float32),
        mesh=mesh,
        compiler_params=pltpu.CompilerParams(needs_layout_passes=False,
                                             use_tc_tiling_on_sc=False),
        scratch_types=[
            pltpu.VMEM((NGC, GCH), jnp.int32),
            pltpu.VMEM((ROWS, D), jnp.float32),
            pltpu.VMEM((ROWS * D,), jnp.float32),
            pltpu.VMEM((L * D,), jnp.float32),
            pltpu.VMEM((D,), jnp.float32),
            pltpu.VMEM((D,), jnp.float32),
            pltpu.SemaphoreType.DMA,
        ],
    )(ids_prep, gene_table, pos_flat, ln_gamma, ln_beta)
    return out_flat.reshape(B, L, D)
